# baseline (speedup 1.0000x reference)
import functools

import numpy as np
import jax
import jax.numpy as jnp
from jax import lax
from jax.experimental import pallas as pl
from jax.experimental.pallas import tpu as pltpu
from jax.experimental.pallas import tpu_sc as plsc

_MAX_LEN = 2048

_NUM_CORES = 2
_NUM_SUBCORES = 16
_NUM_WORKERS = _NUM_CORES * _NUM_SUBCORES
_LANES = 16
_NBUF = 3


def _positional_encoding(max_len, d_model):
    pos = np.arange(max_len, dtype=np.float32)[:, None]
    i2 = np.arange(0, d_model, 2, dtype=np.float32)
    div = np.power(10000.0, i2 / d_model)
    pe = np.zeros((max_len, d_model), dtype=np.float32)
    pe[:, 0::2] = np.sin(pos / div)
    pe[:, 1::2] = np.cos(pos / div)
    return jnp.asarray(pe)


@functools.cache
def _build_kernel(N, S, D, C):
    B = N // S
    W = S // _NUM_WORKERS
    n_chunks = B * W // C
    per_b = W // C
    mesh = plsc.VectorSubcoreMesh(core_axis_name="c", subcore_axis_name="s")

    @functools.partial(
        pl.kernel,
        out_type=jax.ShapeDtypeStruct((N, D), jnp.float32),
        mesh=mesh,
        scratch_types=[
            [pltpu.VMEM((C,), jnp.int32)] * n_chunks,
            pltpu.VMEM((W, D), jnp.float32),
            pltpu.VMEM((_NBUF, C, D), jnp.float32),
            [pltpu.SemaphoreType.DMA] * _NBUF,
            [pltpu.SemaphoreType.DMA] * _NBUF,
        ],
    )
    def emb_kernel(x_hbm, table_hbm, pe_hbm, out_hbm,
                   idx_v, pe_res, rows_v, gsem, ssem):
        wid = lax.axis_index("s") * _NUM_CORES + lax.axis_index("c")
        p0 = wid * W

        pltpu.sync_copy(pe_hbm.at[pl.ds(p0, W)], pe_res)
        for c in range(n_chunks):
            off = (c // per_b) * S + p0 + (c % per_b) * C
            pltpu.sync_copy(x_hbm.at[pl.ds(off, C)], idx_v[c])

        def start_gather(c):
            b = c % _NBUF
            return pltpu.async_copy(table_hbm.at[idx_v[c]], rows_v.at[b], gsem[b])

        inflight = {}
        stores = {}
        for c in range(min(_NBUF - 1, n_chunks)):
            inflight[c] = start_gather(c)
        for c in range(n_chunks):
            b = c % _NBUF
            h = c % per_b
            inflight.pop(c).wait()

            @functools.partial(plsc.parallel_loop, 0, C, unroll=2)
            def row_body(r):
                for j in range(D // _LANES):
                    v = pe_res[h * C + r, pl.ds(j * _LANES, _LANES)]
                    plsc.addupdate(rows_v.at[b, r, pl.ds(j * _LANES, _LANES)], v)

            off = (c // per_b) * S + p0 + h * C
            stores[c] = pltpu.async_copy(
                rows_v.at[b], out_hbm.at[pl.ds(off, C)], ssem[b])
            ahead = c + _NBUF - 1
            if ahead < n_chunks:
                if ahead - _NBUF >= 0:
                    stores[ahead - _NBUF].wait()
                inflight[ahead] = start_gather(ahead)
        for c in range(max(0, n_chunks - _NBUF), n_chunks):
            if c in stores:
                stores[c].wait()

    return emb_kernel


def kernel(x, table):
    B, S = x.shape
    _, D = table.shape
    N = B * S
    pe = _positional_encoding(_MAX_LEN, D)[:S]
    x_flat = x.reshape(N).astype(jnp.int32)
    out = _build_kernel(N, S, D, 32)(x_flat, table, pe)
    return out.reshape(B, S, D)
